# Initial kernel scaffold; baseline (speedup 1.0000x reference)
import jax
import jax.numpy as jnp
from jax.experimental import pallas as pl


def kernel(x, emb):
    raise NotImplementedError("write your pallas kernel here")


# speedup vs baseline: 1.5118x; 1.5118x over previous
import functools

import jax
import jax.numpy as jnp
from jax import lax
from jax.experimental import pallas as pl
from jax.experimental.pallas import tpu as pltpu
from jax.experimental.pallas import tpu_sc as plsc

B, N, C = 8, 1024, 256
K = 8192
M = B * N
BETA = 0.25

TM = 1024
M_TILES = M // TM
TKC = 1024
NKC = K // TKC


def _argmin_body(x_ref, emb_ref, idx_ref):
    x_t = x_ref[...]
    x_sq = jnp.sum(x_t * x_t, axis=1, keepdims=True)
    iota = lax.broadcasted_iota(jnp.int32, (TM, TKC), 1)
    run_min = None
    run_idx = None
    for kc in range(NKC):
        e_t = emb_ref[kc * TKC:(kc + 1) * TKC, :]
        cross = lax.dot_general(x_t, e_t, (((1,), (1,)), ((), ())),
                                preferred_element_type=jnp.float32)
        e_sq = jnp.sum(e_t * e_t, axis=1)[None, :]
        dist2 = jnp.maximum(x_sq + e_sq - 2.0 * cross, 0.0)
        lmin = jnp.min(dist2, axis=1, keepdims=True)
        larg = jnp.min(jnp.where(dist2 == lmin, iota, jnp.int32(K)),
                       axis=1, keepdims=True) + jnp.int32(kc * TKC)
        if run_min is None:
            run_min, run_idx = lmin, larg
        else:
            cond = lmin < run_min
            run_idx = jnp.where(cond, larg, run_idx)
            run_min = jnp.where(cond, lmin, run_min)
    idx_ref[...] = run_idx


def _argmin_call(x2d, emb):
    return pl.pallas_call(
        _argmin_body,
        grid=(M_TILES,),
        in_specs=[
            pl.BlockSpec((TM, C), lambda m: (m, 0)),
            pl.BlockSpec((K, C), lambda m: (0, 0)),
        ],
        out_specs=pl.BlockSpec((TM, 1), lambda m: (m, 0)),
        out_shape=jax.ShapeDtypeStruct((M, 1), jnp.int32),
        compiler_params=pltpu.CompilerParams(
            dimension_semantics=("arbitrary",)),
    )(x2d, emb)


_NC = 2
_NS = 16
_NW = _NC * _NS
ROWS_W = M // _NW


@functools.cache
def _get_sc_gather():
    @functools.partial(
        pl.kernel,
        mesh=plsc.VectorSubcoreMesh(core_axis_name="c", subcore_axis_name="s"),
        out_type=jax.ShapeDtypeStruct((M, C), jnp.float32),
        scratch_types=[
            pltpu.VMEM((ROWS_W,), jnp.int32),
            pltpu.VMEM((ROWS_W, C), jnp.float32),
            pltpu.SemaphoreType.DMA,
        ],
    )
    def _sc_gather(emb_hbm, idx_hbm, out_hbm, idx_v, rows_v, sem):
        wid = lax.axis_index("s") * _NC + lax.axis_index("c")
        base = wid * ROWS_W
        pltpu.sync_copy(idx_hbm.at[pl.ds(base, ROWS_W)], idx_v)
        pltpu.async_copy(emb_hbm.at[idx_v], rows_v, sem).wait()
        pltpu.sync_copy(rows_v, out_hbm.at[pl.ds(base, ROWS_W)])

    return _sc_gather


def _finish_body(x_ref, q_ref, qst_ref, loss_ref):
    m = pl.program_id(0)
    x_t = x_ref[...]
    q_t = q_ref[...]
    d = q_t - x_t
    qst_ref[...] = x_t + d
    ss = jnp.sum(d * d)

    @pl.when(m == 0)
    def _init():
        loss_ref[0, 0] = 0.0

    loss_ref[0, 0] += ss

    @pl.when(m == M_TILES - 1)
    def _scale():
        loss_ref[0, 0] = loss_ref[0, 0] * ((1.0 + BETA) / (M * C))


def _finish_call(x2d, q):
    return pl.pallas_call(
        _finish_body,
        grid=(M_TILES,),
        in_specs=[
            pl.BlockSpec((TM, C), lambda m: (m, 0)),
            pl.BlockSpec((TM, C), lambda m: (m, 0)),
        ],
        out_specs=[
            pl.BlockSpec((TM, C), lambda m: (m, 0)),
            pl.BlockSpec(memory_space=pltpu.SMEM,
                         block_shape=(1, 1), index_map=lambda m: (0, 0)),
        ],
        out_shape=[
            jax.ShapeDtypeStruct((M, C), jnp.float32),
            jax.ShapeDtypeStruct((1, 1), jnp.float32),
        ],
        compiler_params=pltpu.CompilerParams(
            dimension_semantics=("arbitrary",)),
    )(x2d, q)


def kernel(x, emb):
    x2d = x.reshape(M, C)
    idx2 = _argmin_call(x2d, emb)
    idx = idx2.reshape(M)
    q = _get_sc_gather()(emb, idx)
    qst2d, loss = _finish_call(x2d, q)
    return qst2d.reshape(B, N, C), loss[0, 0], idx.reshape(B, N)
